# Initial kernel scaffold; baseline (speedup 1.0000x reference)
import jax
import jax.numpy as jnp
from jax.experimental import pallas as pl


def kernel(context_ids, doc_ids, target_noise_ids, D, W, O):
    raise NotImplementedError("write your pallas kernel here")


# speedup vs baseline: 1.9522x; 1.9522x over previous
import functools

import jax
import jax.numpy as jnp
from jax import lax
from jax.experimental import pallas as pl
from jax.experimental.pallas import tpu as pltpu
from jax.experimental.pallas import tpu_sc as plsc

B = 4096
CTX = 20
NOISE = 26
VD = 128
NW = 32
NB = B // NW
SUB = 32
NSUB = NB // SUB

_MESH = plsc.VectorSubcoreMesh(core_axis_name="c", subcore_axis_name="s")


@functools.partial(
    pl.kernel,
    mesh=_MESH,
    out_type=jax.ShapeDtypeStruct((B, VD), jnp.float32),
    scratch_types=[
        pltpu.VMEM((1, 128), jnp.int32),
        pltpu.VMEM((CTX, 128), jnp.int32),
        pltpu.VMEM((NB, VD), jnp.float32),
        pltpu.VMEM((SUB * CTX, VD), jnp.float32),
        pltpu.VMEM((SUB, VD), jnp.float32),
        pltpu.SemaphoreType.DMA,
        pltpu.SemaphoreType.DMA,
    ],
)
def _sc_pool(doc3_hbm, ctx3_hbm, d_hbm, w_hbm, x_hbm,
             didx_v, cidx_v, drows_v, wrows_v, acc_v, dsem, wsem):
    wid = lax.axis_index("s") * 2 + lax.axis_index("c")
    base = wid * NB
    pltpu.sync_copy(doc3_hbm.at[wid], didx_v)
    dcp = pltpu.async_copy(d_hbm.at[didx_v.at[0]], drows_v, dsem)
    pltpu.sync_copy(ctx3_hbm.at[wid], cidx_v)
    dcp.wait()

    @pl.loop(0, NSUB)
    def _(s):
        cps = [
            pltpu.async_copy(
                w_hbm.at[cidx_v.at[s * 5 + j]],
                wrows_v.at[pl.ds(j * 128, 128)],
                wsem,
            )
            for j in range(5)
        ]
        for cp in cps:
            cp.wait()

        @pl.loop(0, SUB)
        def _(b):
            for c in range(VD // 16):
                sl = pl.ds(c * 16, 16)
                v = drows_v[s * SUB + b, sl]
                for j in range(CTX):
                    v = v + wrows_v[b * CTX + j, sl]
                acc_v[b, sl] = v

        pltpu.sync_copy(acc_v, x_hbm.at[pl.ds(base + s * SUB, SUB)])


_NG = B * NOISE
_GPW = _NG // NW
_GSTEPS = _GPW // 128


@functools.partial(
    pl.kernel,
    mesh=_MESH,
    out_type=jax.ShapeDtypeStruct((_NG, VD), jnp.float32),
    scratch_types=[
        pltpu.VMEM((_GSTEPS, 128), jnp.int32),
        pltpu.VMEM((128, VD), jnp.float32),
        pltpu.SemaphoreType.DMA,
    ],
)
def _sc_gather_o(ot_hbm, tid3_hbm, og_hbm, tidx_v, rows_v, sem):
    wid = lax.axis_index("s") * 2 + lax.axis_index("c")
    base = wid * _GPW
    pltpu.sync_copy(tid3_hbm.at[wid], tidx_v)

    @pl.loop(0, _GSTEPS)
    def _(ci):
        pltpu.async_copy(ot_hbm.at[tidx_v.at[ci]], rows_v, sem).wait()
        pltpu.sync_copy(rows_v, og_hbm.at[pl.ds(base + ci * 128, 128)])


_TRB = 1024
_NTR = -(-100000 // _TRB)
_OTROWS = _NTR * _TRB


def _tr_body(o_ref, ot_ref):
    ot_ref[...] = o_ref[...].T


def _score_body(x_ref, og_ref, s_ref):
    x = x_ref[...]
    og = og_ref[...]
    s_ref[...] = jnp.sum(og * x[:, None, :], axis=-1)


def kernel(context_ids, doc_ids, target_noise_ids, D, W, O):
    nw = O.shape[1]
    ctx3 = context_ids.astype(jnp.int32).reshape(NW, NB * CTX // 128, 128)
    doc3 = doc_ids.astype(jnp.int32).reshape(NW, 1, 128)
    tid3 = target_noise_ids.astype(jnp.int32).reshape(NW, _GSTEPS, 128)

    x = _sc_pool(doc3, ctx3, D, W)

    ot = pl.pallas_call(
        _tr_body,
        grid=(_NTR,),
        in_specs=[pl.BlockSpec((VD, _TRB), lambda i: (0, i))],
        out_specs=pl.BlockSpec((_TRB, VD), lambda i: (i, 0)),
        out_shape=jax.ShapeDtypeStruct((_OTROWS, VD), jnp.float32),
    )(O)

    og = _sc_gather_o(ot, tid3)

    scores = pl.pallas_call(
        _score_body,
        grid=(B // 256,),
        in_specs=[
            pl.BlockSpec((256, VD), lambda i: (i, 0)),
            pl.BlockSpec((256, NOISE, VD), lambda i: (i, 0, 0)),
        ],
        out_specs=pl.BlockSpec((256, NOISE), lambda i: (i, 0)),
        out_shape=jax.ShapeDtypeStruct((B, NOISE), jnp.float32),
    )(x, og.reshape(B, NOISE, VD))

    return scores


# speedup vs baseline: 2.2873x; 1.1716x over previous
import functools

import jax
import jax.numpy as jnp
from jax import lax
from jax.experimental import pallas as pl
from jax.experimental.pallas import tpu as pltpu
from jax.experimental.pallas import tpu_sc as plsc

B = 4096
CTX = 20
NOISE = 26
VD = 128
NW = 32
NB = B // NW
SUB = 16
NSUB = NB // SUB
IROW = 64

_MESH = plsc.VectorSubcoreMesh(core_axis_name="c", subcore_axis_name="s")


@functools.partial(
    pl.kernel,
    mesh=_MESH,
    out_type=jax.ShapeDtypeStruct((B, VD), jnp.float32),
    scratch_types=[
        pltpu.VMEM((1, 128), jnp.int32),
        pltpu.VMEM((NB * CTX // IROW, IROW), jnp.int32),
        pltpu.VMEM((NB, VD), jnp.float32),
        pltpu.VMEM((SUB * CTX, VD), jnp.float32),
        pltpu.VMEM((SUB * CTX, VD), jnp.float32),
        pltpu.VMEM((SUB, VD), jnp.float32),
        pltpu.SemaphoreType.DMA,
        pltpu.SemaphoreType.DMA,
        pltpu.SemaphoreType.DMA,
    ],
)
def _sc_pool(doc3_hbm, ctx3_hbm, d_hbm, w_hbm, x_hbm,
             didx_v, cidx_v, drows_v, wrows_a, wrows_b, acc_v,
             dsem, wsem_a, wsem_b):
    wid = lax.axis_index("s") * 2 + lax.axis_index("c")
    base = wid * NB
    pltpu.sync_copy(doc3_hbm.at[wid], didx_v)
    dcp = pltpu.async_copy(d_hbm.at[didx_v.at[0]], drows_v, dsem)
    pltpu.sync_copy(ctx3_hbm.at[wid], cidx_v)

    def fire(s, buf, sem):
        for j in range(5):
            pltpu.async_copy(
                w_hbm.at[cidx_v.at[s * 5 + j]],
                buf.at[pl.ds(j * IROW, IROW)],
                sem,
            )

    def drain(s, buf, sem):
        for j in range(5):
            pltpu.make_async_copy(
                w_hbm.at[cidx_v.at[s * 5 + j]],
                buf.at[pl.ds(j * IROW, IROW)],
                sem,
            ).wait()

    fire(0, wrows_a, wsem_a)
    fire(1, wrows_b, wsem_b)
    dcp.wait()

    @pl.loop(0, NSUB // 2)
    def _(g):
        for p, (buf, sem) in enumerate(((wrows_a, wsem_a), (wrows_b, wsem_b))):
            s = 2 * g + p
            drain(s, buf, sem)

            @pl.loop(0, SUB)
            def _(b):
                for c in range(VD // 16):
                    sl = pl.ds(c * 16, 16)
                    v = drows_v[s * SUB + b, sl]
                    for j in range(CTX):
                        v = v + buf[b * CTX + j, sl]
                    acc_v[b, sl] = v

            pltpu.sync_copy(acc_v, x_hbm.at[pl.ds(base + s * SUB, SUB)])

            @pl.when(s + 2 < NSUB)
            def _():
                fire(s + 2, buf, sem)


_NG = B * NOISE
_GPW = _NG // NW
_GSTEPS = _GPW // 128


@functools.partial(
    pl.kernel,
    mesh=_MESH,
    out_type=jax.ShapeDtypeStruct((_NG, VD), jnp.float32),
    scratch_types=[
        pltpu.VMEM((_GSTEPS, 128), jnp.int32),
        pltpu.VMEM((128, VD), jnp.float32),
        pltpu.VMEM((128, VD), jnp.float32),
        pltpu.SemaphoreType.DMA,
        pltpu.SemaphoreType.DMA,
    ],
)
def _sc_gather_o(ot_hbm, tid3_hbm, og_hbm, tidx_v, rows_a, rows_b,
                 gsem_a, gsem_b):
    wid = lax.axis_index("s") * 2 + lax.axis_index("c")
    base = wid * _GPW
    pltpu.sync_copy(tid3_hbm.at[wid], tidx_v)

    pltpu.async_copy(ot_hbm.at[tidx_v.at[0]], rows_a, gsem_a)
    pltpu.async_copy(ot_hbm.at[tidx_v.at[1]], rows_b, gsem_b)

    @pl.loop(0, _GSTEPS // 2)
    def _(g):
        for p, (buf, sem) in enumerate(((rows_a, gsem_a), (rows_b, gsem_b))):
            ci = 2 * g + p
            pltpu.make_async_copy(ot_hbm.at[tidx_v.at[ci]], buf, sem).wait()
            pltpu.sync_copy(buf, og_hbm.at[pl.ds(base + ci * 128, 128)])

            @pl.when(ci + 2 < _GSTEPS)
            def _():
                pltpu.async_copy(ot_hbm.at[tidx_v.at[ci + 2]], buf, sem)


_TRB = 1024
_NTR = -(-100000 // _TRB)
_OTROWS = _NTR * _TRB


def _tr_body(o_ref, ot_ref):
    ot_ref[...] = o_ref[...].T


def _score_body(x_ref, og_ref, s_ref):
    x = x_ref[...]
    og = og_ref[...]
    s = jnp.sum(og * x[None, :, :], axis=-1)
    s_ref[...] = s.T


def kernel(context_ids, doc_ids, target_noise_ids, D, W, O):
    ctx3 = context_ids.astype(jnp.int32).reshape(NW, NB * CTX // IROW, IROW)
    doc3 = doc_ids.astype(jnp.int32).reshape(NW, 1, 128)
    tid3 = target_noise_ids.astype(jnp.int32).T.reshape(NW, _GSTEPS, 128)

    x = _sc_pool(doc3, ctx3, D, W)

    ot = pl.pallas_call(
        _tr_body,
        grid=(_NTR,),
        in_specs=[pl.BlockSpec((VD, _TRB), lambda i: (0, i))],
        out_specs=pl.BlockSpec((_TRB, VD), lambda i: (i, 0)),
        out_shape=jax.ShapeDtypeStruct((_OTROWS, VD), jnp.float32),
    )(O)

    og = _sc_gather_o(ot, tid3)

    scores = pl.pallas_call(
        _score_body,
        grid=(B // 256,),
        in_specs=[
            pl.BlockSpec((256, VD), lambda i: (i, 0)),
            pl.BlockSpec((NOISE, 256, VD), lambda i: (0, i, 0)),
        ],
        out_specs=pl.BlockSpec((256, NOISE), lambda i: (i, 0)),
        out_shape=jax.ShapeDtypeStruct((B, NOISE), jnp.float32),
    )(x, og.reshape(NOISE, B, VD))

    return scores


# speedup vs baseline: 5.4869x; 2.3988x over previous
import functools

import jax
import jax.numpy as jnp
from jax import lax
from jax.experimental import pallas as pl
from jax.experimental.pallas import tpu as pltpu
from jax.experimental.pallas import tpu_sc as plsc

B = 4096
CTX = 20
NOISE = 26
VD = 128
NW = 32
NB = B // NW
SUB = 8
NSUB = NB // SUB
IROW = 32
CROWS = NB * CTX // IROW

_NG = B * NOISE
_GPW = _NG // NW
GC = 104
NOGB = 4

_MESH = plsc.VectorSubcoreMesh(core_axis_name="c", subcore_axis_name="s")


@functools.partial(
    pl.kernel,
    mesh=_MESH,
    out_type=[
        jax.ShapeDtypeStruct((B, VD), jnp.float32),
        jax.ShapeDtypeStruct((_NG, VD), jnp.float32),
    ],
    scratch_types=[
        pltpu.VMEM((1, 128), jnp.int32),
        pltpu.VMEM((CROWS, IROW), jnp.int32),
        pltpu.VMEM((_GPW // GC, GC), jnp.int32),
        pltpu.VMEM((NB, VD), jnp.float32),
        pltpu.VMEM((SUB * CTX, VD), jnp.float32),
        pltpu.VMEM((SUB * CTX, VD), jnp.float32),
        pltpu.VMEM((SUB, VD), jnp.float32),
        pltpu.VMEM((GC, VD), jnp.float32),
        pltpu.VMEM((GC, VD), jnp.float32),
        pltpu.VMEM((GC, VD), jnp.float32),
        pltpu.VMEM((GC, VD), jnp.float32),
        pltpu.SemaphoreType.DMA,
        pltpu.SemaphoreType.DMA,
        pltpu.SemaphoreType.DMA,
        pltpu.SemaphoreType.DMA,
        pltpu.SemaphoreType.DMA,
        pltpu.SemaphoreType.DMA,
        pltpu.SemaphoreType.DMA,
        pltpu.SemaphoreType.DMA,
        pltpu.SemaphoreType.DMA,
        pltpu.SemaphoreType.DMA,
        pltpu.SemaphoreType.DMA,
    ],
)
def _sc_fused(doc3_hbm, ctx3_hbm, tid3_hbm, d_hbm, w_hbm, ot_hbm,
              x_hbm, og_hbm,
              didx_v, cidx_v, tidx_v, drows_v, wrows_a, wrows_b, acc_v,
              og0, og1, og2, og3,
              dsem, wsem_a, wsem_b,
              osem0, osem1, osem2, osem3,
              owsem0, owsem1, owsem2, owsem3):
    wid = lax.axis_index("s") * 2 + lax.axis_index("c")
    base = wid * NB
    og_base = wid * _GPW
    ogbuf = (og0, og1, og2, og3)
    osem = (osem0, osem1, osem2, osem3)
    owsem = (owsem0, owsem1, owsem2, owsem3)
    wrows = (wrows_a, wrows_b)
    wsem = (wsem_a, wsem_b)

    def w_fire(s, p):
        for j in range(5):
            pltpu.async_copy(
                w_hbm.at[cidx_v.at[s * 5 + j]],
                wrows[p].at[pl.ds(j * IROW, IROW)],
                wsem[p],
            )

    def w_drain(s, p):
        for j in range(5):
            pltpu.make_async_copy(
                w_hbm.at[cidx_v.at[s * 5 + j]],
                wrows[p].at[pl.ds(j * IROW, IROW)],
                wsem[p],
            ).wait()

    def og_fire(c, b):
        pltpu.async_copy(ot_hbm.at[tidx_v.at[c]], ogbuf[b], osem[b])

    def og_gather_drain(c, b):
        pltpu.make_async_copy(ot_hbm.at[tidx_v.at[c]], ogbuf[b], osem[b]).wait()

    def og_write(c, b):
        pltpu.async_copy(ogbuf[b], og_hbm.at[pl.ds(og_base + c * GC, GC)],
                         owsem[b])

    def og_write_drain(c, b):
        pltpu.make_async_copy(ogbuf[b], og_hbm.at[pl.ds(og_base + c * GC, GC)],
                              owsem[b]).wait()

    pltpu.sync_copy(doc3_hbm.at[wid], didx_v)
    dcp = pltpu.async_copy(d_hbm.at[didx_v.at[0]], drows_v, dsem)
    pltpu.sync_copy(ctx3_hbm.at[wid], cidx_v)
    pltpu.sync_copy(tid3_hbm.at[wid], tidx_v)

    w_fire(0, 0)
    w_fire(1, 1)
    for c in range(NOGB):
        og_fire(c, c)
    dcp.wait()

    @pl.loop(0, NSUB // 2)
    def _(g):
        for p in range(2):
            s = 2 * g + p
            for q in range(2):
                c = 2 * s + q
                bq = (2 * p + q) % NOGB
                og_gather_drain(c, bq)
                og_write(c, bq)

            w_drain(s, p)

            @pl.loop(0, SUB)
            def _(b):
                for ch in range(VD // 16):
                    sl = pl.ds(ch * 16, 16)
                    v = drows_v[s * SUB + b, sl]
                    for j in range(CTX):
                        v = v + wrows[p][b * CTX + j, sl]
                    acc_v[b, sl] = v

            pltpu.sync_copy(acc_v, x_hbm.at[pl.ds(base + s * SUB, SUB)])

            for q in range(2):
                cn = 2 * s + 4 + q
                bq = (2 * p + q) % NOGB

                @pl.when(cn < _GPW // GC)
                def _():
                    og_write_drain(cn - NOGB, bq)
                    og_fire(cn, bq)

            @pl.when(s + 2 < NSUB)
            def _():
                w_fire(s + 2, p)

    for b in range(NOGB):
        og_write_drain(_GPW // GC - NOGB + b, b)


def _score_body(x_ref, og_ref, s_ref):
    x = x_ref[...]
    og = og_ref[...]
    s_ref[...] = jnp.sum(og * x[None, :, :], axis=-1)


def kernel(context_ids, doc_ids, target_noise_ids, D, W, O):
    ctx3 = context_ids.astype(jnp.int32).reshape(NW, CROWS, IROW)
    doc3 = doc_ids.astype(jnp.int32).reshape(NW, 1, 128)
    tid3 = target_noise_ids.astype(jnp.int32).T.reshape(NW, _GPW // GC, GC)

    ot = jnp.transpose(O)

    x, og = _sc_fused(doc3, ctx3, tid3, D, W, ot)

    scores_t = pl.pallas_call(
        _score_body,
        grid=(B // 512,),
        in_specs=[
            pl.BlockSpec((512, VD), lambda i: (i, 0)),
            pl.BlockSpec((NOISE, 512, VD), lambda i: (0, i, 0)),
        ],
        out_specs=pl.BlockSpec((NOISE, 512), lambda i: (0, i)),
        out_shape=jax.ShapeDtypeStruct((NOISE, B), jnp.float32),
    )(x, og.reshape(NOISE, B, VD))

    return jnp.transpose(scores_t)
